# baseline (speedup 1.0000x reference)
import functools

import jax
import jax.numpy as jnp
from jax import lax
from jax.experimental import pallas as pl
from jax.experimental.pallas import tpu as pltpu
from jax.experimental.pallas import tpu_sc as plsc

_N_ROWS = 16384
_N_CLASSES = 1000
_EPS = 0.001
_ON = 1.0 / _N_CLASSES + _EPS
_OFF = 1.0 / _N_CLASSES - _EPS / (_N_CLASSES - 1)

_N_WORKERS = 32
_ROWS_PER_WORKER = _N_ROWS // _N_WORKERS
_BATCH = 16
_N_BATCHES = _ROWS_PER_WORKER // _BATCH
_N_PAR = 4
_N_STEPS = _N_BATCHES // _N_PAR
_FULL_CHUNKS = _N_CLASSES // 16
_TAIL_OFF = _N_CLASSES - 16


def _sc_body(in_hbm, out_hbm,
             in0, in1, in2, in3, ob0, ob1, ob2, ob3,
             si0, si1, si2, si3, so0, so1, so2, so3):
    wid = lax.axis_index("s") * 2 + lax.axis_index("c")
    row0 = wid * _ROWS_PER_WORKER

    inbufs = (in0, in1, in2, in3)
    outbufs = (ob0, ob1, ob2, ob3)
    isems = (si0, si1, si2, si3)
    osems = (so0, so1, so2, so3)

    lane = lax.iota(jnp.int32, 16)
    off_vec = jnp.full((16,), _OFF, jnp.float32)
    on_vec = jnp.full((16,), _ON, jnp.float32)
    ninf = jnp.full((16,), -jnp.inf, jnp.float32)
    zeros_i = jnp.zeros((16,), jnp.int32)

    base_k = tuple(lane + 16 * k for k in range(4))
    ones_i = jnp.ones((16,), jnp.int32)
    big_i = jnp.full((16,), _N_CLASSES, jnp.int32)

    def _merge(mv_a, ci_a, mv_b, ci_b):
        take_b = (mv_b > mv_a) | ((mv_b == mv_a) & (ci_b < ci_a))
        return jnp.where(take_b, mv_b, mv_a), jnp.where(take_b, ci_b, ci_a)

    def _argmax_group(inb, g):
        def _row(r, acc):
            rr = g * 16 + r

            def _step(t, carry):
                tv, mv0, mi0, mv1, mi1, mv2, mi2, mv3, mi3 = carry
                o = t * 64
                x0 = inb[rr, pl.ds(o, 16)]
                x1 = inb[rr, pl.ds(o + 16, 16)]
                x2 = inb[rr, pl.ds(o + 32, 16)]
                x3 = inb[rr, pl.ds(o + 48, 16)]
                g0 = x0 > mv0
                g1 = x1 > mv1
                g2 = x2 > mv2
                g3 = x3 > mv3
                return (tv + ones_i,
                        jnp.where(g0, x0, mv0), jnp.where(g0, tv, mi0),
                        jnp.where(g1, x1, mv1), jnp.where(g1, tv, mi1),
                        jnp.where(g2, x2, mv2), jnp.where(g2, tv, mi2),
                        jnp.where(g3, x3, mv3), jnp.where(g3, tv, mi3))

            init = (zeros_i,
                    ninf, zeros_i, ninf, zeros_i,
                    ninf, zeros_i, ninf, zeros_i)
            _, mv0, mi0, mv1, mi1, mv2, mi2, mv3, mi3 = lax.fori_loop(
                0, 15, _step, init)

            c0 = (mi0 << 6) + base_k[0]
            c1 = (mi1 << 6) + base_k[1]
            c2 = (mi2 << 6) + base_k[2]
            c3 = (mi3 << 6) + base_k[3]
            mva, cia = _merge(mv0, c0, mv1, c1)
            mvb, cib = _merge(mv2, c2, mv3, c3)
            mv, ci = _merge(mva, cia, mvb, cib)

            for off in (960, 976, _TAIL_OFF):
                x = inb[rr, pl.ds(off, 16)]
                gt = x > mv
                mv = jnp.where(gt, x, mv)
                ci = jnp.where(gt, off + lane, ci)

            m = jnp.max(mv)
            a = jnp.min(jnp.where(mv == m, ci, big_i))
            return jnp.where(lane == r, a, acc)

        return lax.fori_loop(0, 16, _row, zeros_i)

    for j in range(_N_PAR):
        pltpu.async_copy(
            in_hbm.at[pl.ds(row0 + j * _BATCH, _BATCH)], inbufs[j], isems[j])

    for ob in outbufs:
        def _fill_row(r, _, ob=ob):
            for c in range(_FULL_CHUNKS):
                ob[r, pl.ds(c * 16, 16)] = off_vec
            ob[r, pl.ds(_TAIL_OFF, 16)] = off_vec
            return _
        lax.fori_loop(0, _BATCH, _fill_row, None)

    for j in range(_N_PAR):
        pltpu.async_copy(
            outbufs[j], out_hbm.at[pl.ds(row0 + j * _BATCH, _BATCH)], osems[j])

    def _one(inb, ob, isem, osem, b, pc):
        cur = row0 + b * _BATCH
        prev = row0 + jnp.maximum(b - _N_PAR, 0) * _BATCH
        nxt = row0 + jnp.minimum(b + _N_PAR, _N_BATCHES - 1) * _BATCH

        pltpu.make_async_copy(ob, out_hbm.at[pl.ds(prev, _BATCH)], osem).wait()
        plsc.store_scatter(ob, [lane, pc], off_vec)

        pltpu.make_async_copy(in_hbm.at[pl.ds(cur, _BATCH)], inb, isem).wait()
        cols = _argmax_group(inb, 0)
        pltpu.async_copy(in_hbm.at[pl.ds(nxt, _BATCH)], inb, isem)

        plsc.store_scatter(ob, [lane, cols], on_vec)
        pltpu.async_copy(ob, out_hbm.at[pl.ds(cur, _BATCH)], osem)
        return cols

    def _step4(t, carry):
        return tuple(
            _one(inbufs[j], outbufs[j], isems[j], osems[j],
                 _N_PAR * t + j, carry[j])
            for j in range(_N_PAR))

    lax.fori_loop(0, _N_STEPS, _step4, (zeros_i,) * _N_PAR)

    lastb = row0 + (_N_BATCHES - 1) * _BATCH
    for j in range(_N_PAR):
        lastw = row0 + (_N_BATCHES - _N_PAR + j) * _BATCH
        pltpu.make_async_copy(
            outbufs[j], out_hbm.at[pl.ds(lastw, _BATCH)], osems[j]).wait()
        pltpu.make_async_copy(
            in_hbm.at[pl.ds(lastb, _BATCH)], inbufs[j], isems[j]).wait()


def kernel(input):
    mesh = plsc.VectorSubcoreMesh(core_axis_name="c", subcore_axis_name="s")
    fn = functools.partial(
        pl.kernel,
        out_type=jax.ShapeDtypeStruct((_N_ROWS, _N_CLASSES), jnp.float32),
        mesh=mesh,
        scratch_types=(
            [pltpu.VMEM((_BATCH, _N_CLASSES), jnp.float32)] * 8
            + [pltpu.SemaphoreType.DMA] * 8
        ),
        compiler_params=pltpu.CompilerParams(needs_layout_passes=False),
    )(_sc_body)
    return fn(input)


# speedup vs baseline: 1.0057x; 1.0057x over previous
import functools

import jax
import jax.numpy as jnp
from jax import lax
from jax.experimental import pallas as pl
from jax.experimental.pallas import tpu as pltpu
from jax.experimental.pallas import tpu_sc as plsc

_N_ROWS = 16384
_N_CLASSES = 1000
_EPS = 0.001
_ON = 1.0 / _N_CLASSES + _EPS
_OFF = 1.0 / _N_CLASSES - _EPS / (_N_CLASSES - 1)

_N_WORKERS = 32
_ROWS_PER_WORKER = _N_ROWS // _N_WORKERS
_BATCH = 16
_N_BATCHES = _ROWS_PER_WORKER // _BATCH
_N_PAR = 4
_N_STEPS = _N_BATCHES // _N_PAR
_FULL_CHUNKS = _N_CLASSES // 16
_TAIL_OFF = _N_CLASSES - 16


def _sc_body(in_hbm, out_hbm,
             in0, in1, in2, in3, ob0, ob1, ob2, ob3,
             si0, si1, si2, si3, so0, so1, so2, so3):
    wid = lax.axis_index("s") * 2 + lax.axis_index("c")
    row0 = wid * _ROWS_PER_WORKER

    inbufs = (in0, in1, in2, in3)
    outbufs = (ob0, ob1, ob2, ob3)
    isems = (si0, si1, si2, si3)
    osems = (so0, so1, so2, so3)

    lane = lax.iota(jnp.int32, 16)
    off_vec = jnp.full((16,), _OFF, jnp.float32)
    on_vec = jnp.full((16,), _ON, jnp.float32)
    ninf = jnp.full((16,), -jnp.inf, jnp.float32)
    zeros_i = jnp.zeros((16,), jnp.int32)

    base_k = tuple(lane + 16 * k for k in range(4))
    ones_i = jnp.ones((16,), jnp.int32)
    big_i = jnp.full((16,), _N_CLASSES, jnp.int32)

    def _merge(mv_a, ci_a, mv_b, ci_b):
        take_b = (mv_b > mv_a) | ((mv_b == mv_a) & (ci_b < ci_a))
        return jnp.where(take_b, mv_b, mv_a), jnp.where(take_b, ci_b, ci_a)

    def _argmax_group(inb, g):
        def _row(r, acc):
            rr = g * 16 + r

            def _step(t, carry):
                tv, mv0, mi0, mv1, mi1, mv2, mi2, mv3, mi3 = carry
                o = t * 64
                x0 = inb[rr, pl.ds(o, 16)]
                x1 = inb[rr, pl.ds(o + 16, 16)]
                x2 = inb[rr, pl.ds(o + 32, 16)]
                x3 = inb[rr, pl.ds(o + 48, 16)]
                g0 = x0 > mv0
                g1 = x1 > mv1
                g2 = x2 > mv2
                g3 = x3 > mv3
                return (tv + ones_i,
                        jnp.where(g0, x0, mv0), jnp.where(g0, tv, mi0),
                        jnp.where(g1, x1, mv1), jnp.where(g1, tv, mi1),
                        jnp.where(g2, x2, mv2), jnp.where(g2, tv, mi2),
                        jnp.where(g3, x3, mv3), jnp.where(g3, tv, mi3))

            init = (zeros_i,
                    ninf, zeros_i, ninf, zeros_i,
                    ninf, zeros_i, ninf, zeros_i)
            _, mv0, mi0, mv1, mi1, mv2, mi2, mv3, mi3 = lax.fori_loop(
                0, 15, _step, init)

            c0 = (mi0 << 6) + base_k[0]
            c1 = (mi1 << 6) + base_k[1]
            c2 = (mi2 << 6) + base_k[2]
            c3 = (mi3 << 6) + base_k[3]
            mva, cia = _merge(mv0, c0, mv1, c1)
            mvb, cib = _merge(mv2, c2, mv3, c3)
            mv, ci = _merge(mva, cia, mvb, cib)

            for off in (960, 976, _TAIL_OFF):
                x = inb[rr, pl.ds(off, 16)]
                gt = x > mv
                mv = jnp.where(gt, x, mv)
                ci = jnp.where(gt, off + lane, ci)

            m = jnp.max(mv)
            a = jnp.min(jnp.where(mv == m, ci, big_i))
            return jnp.where(lane == r, a, acc)

        return lax.fori_loop(0, 16, _row, zeros_i)

    for j in range(_N_PAR):
        pltpu.async_copy(
            in_hbm.at[pl.ds(row0 + j * _BATCH, _BATCH)], inbufs[j], isems[j])

    for ob in outbufs:
        def _fill_row(r, _, ob=ob):
            for c in range(_FULL_CHUNKS):
                ob[r, pl.ds(c * 16, 16)] = off_vec
            ob[r, pl.ds(_TAIL_OFF, 16)] = off_vec
            return _
        lax.fori_loop(0, _BATCH, _fill_row, None)

    for j in range(_N_PAR):
        pltpu.async_copy(
            outbufs[j], out_hbm.at[pl.ds(row0 + j * _BATCH, _BATCH)], osems[j])

    def _one(inb, ob, isem, osem, b, pc):
        cur = row0 + b * _BATCH
        prev = row0 + jnp.maximum(b - _N_PAR, 0) * _BATCH
        nxt = row0 + jnp.minimum(b + _N_PAR, _N_BATCHES - 1) * _BATCH

        pltpu.make_async_copy(ob, out_hbm.at[pl.ds(prev, _BATCH)], osem).wait()
        plsc.store_scatter(ob, [lane, pc], off_vec)

        pltpu.make_async_copy(in_hbm.at[pl.ds(cur, _BATCH)], inb, isem).wait()
        cols = _argmax_group(inb, 0)
        pltpu.async_copy(in_hbm.at[pl.ds(nxt, _BATCH)], inb, isem)

        plsc.store_scatter(ob, [lane, cols], on_vec)
        pltpu.async_copy(ob, out_hbm.at[pl.ds(cur, _BATCH)], osem)
        return cols

    def _step4(t, carry):
        return tuple(
            _one(inbufs[j], outbufs[j], isems[j], osems[j],
                 _N_PAR * t + j, carry[j])
            for j in range(_N_PAR))

    lax.fori_loop(0, _N_STEPS, _step4, (zeros_i,) * _N_PAR)

    lastb = row0 + (_N_BATCHES - 1) * _BATCH
    for j in range(_N_PAR):
        lastw = row0 + (_N_BATCHES - _N_PAR + j) * _BATCH
        pltpu.make_async_copy(
            outbufs[j], out_hbm.at[pl.ds(lastw, _BATCH)], osems[j]).wait()
        pltpu.make_async_copy(
            in_hbm.at[pl.ds(lastb, _BATCH)], inbufs[j], isems[j]).wait()


def kernel(input):
    mesh = plsc.VectorSubcoreMesh(core_axis_name="c", subcore_axis_name="s")
    fn = functools.partial(
        pl.kernel,
        out_type=jax.ShapeDtypeStruct((_N_ROWS, _N_CLASSES), jnp.float32),
        mesh=mesh,
        scratch_types=(
            [pltpu.VMEM((_BATCH, _N_CLASSES), jnp.float32)] * 8
            + [pltpu.SemaphoreType.DMA] * 8
        ),
        compiler_params=pltpu.CompilerParams(
            needs_layout_passes=False, skip_device_barrier=True),
    )(_sc_body)
    return fn(input)
